# baseline (speedup 1.0000x reference)
import jax
import jax.numpy as jnp
from jax import lax
from jax.experimental import pallas as pl
from jax.experimental.pallas import tpu as pltpu
from jax.experimental.pallas import tpu_sc as plsc

NUM_USERS = 100000
NUM_ITEMS = 1000000
K = 16
H = 20
B = 4096
R = 8
L = 128

NC = 2
NS = 16
NW = NC * NS
CH = B // NW
CHR = CH // R


def _compact(fat, slots, dst):
  def blk(b, c):
    e_vec = b * 16 + lax.iota(jnp.int32, 16)
    s_vec = slots[pl.ds(b * 16, 16)] * 16
    def kstep(k, c2):
      v = plsc.load_gather(fat, [e_vec, s_vec + k])
      flat = e_vec * 16 + k
      plsc.store_scatter(dst, [lax.shift_right_logical(flat, 7),
                               jnp.bitwise_and(flat, 127)], v)
      return c2
    return lax.fori_loop(0, K, kstep, c, unroll=True)
  lax.fori_loop(0, CH // 16, blk, 0, unroll=True)


def _sc_body(users_h, items_h, gur_tab, gir_tab, ui1_h,
             gu_o, gihist_o, gmi_o,
             users_v, items_v, idx20, idxj, histcol, rowidx, slots,
             fat, packed, sem):
  wid = lax.axis_index("s") * NC + lax.axis_index("c")
  base = wid * CH
  obase = wid * CHR

  pltpu.sync_copy(users_h.at[pl.ds(base, CH)], users_v)
  pltpu.sync_copy(items_h.at[pl.ds(base, CH)], items_v)

  def fat_gather_pack(tab, ids_ref, out, orow):
    def prep(t, c):
      sl = pl.ds(t * 16, 16)
      ids = ids_ref[sl]
      rowidx[sl] = lax.shift_right_logical(ids, 3)
      slots[sl] = jnp.bitwise_and(ids, 7)
      return c
    lax.fori_loop(0, CH // 16, prep, 0, unroll=True)
    pltpu.async_copy(tab.at[rowidx], fat, sem).wait()
    _compact(fat, slots, packed)
    pltpu.sync_copy(packed, out)

  fat_gather_pack(gur_tab, users_v, gu_o.at[pl.ds(obase, CHR)], obase)
  fat_gather_pack(gir_tab, items_v, gmi_o.at[pl.ds(obase, CHR)], obase)

  def mul_body(t, c):
    sl = pl.ds(t * 16, 16)
    idx20[sl] = users_v[sl] * H
    return c
  lax.fori_loop(0, CH // 16, mul_body, 0, unroll=True)

  def j_body(j, c):
    def add_body(t, c2):
      sl = pl.ds(t * 16, 16)
      idxj[sl] = idx20[sl] + j
      return c2
    lax.fori_loop(0, CH // 16, add_body, 0, unroll=True)
    pltpu.async_copy(ui1_h.at[idxj], histcol, sem).wait()
    fat_gather_pack(gir_tab, histcol, gihist_o.at[j, pl.ds(obase, CHR)], obase)
    return c
  lax.fori_loop(0, H, j_body, 0)


def _sc_gather(users, items, Gu_r, Gi_r, ui1):
  mesh = plsc.VectorSubcoreMesh(core_axis_name="c", subcore_axis_name="s")
  f = pl.kernel(
      _sc_body,
      out_type=[
          jax.ShapeDtypeStruct((B // R, L), jnp.float32),
          jax.ShapeDtypeStruct((H, B // R, L), jnp.float32),
          jax.ShapeDtypeStruct((B // R, L), jnp.float32),
      ],
      mesh=mesh,
      scratch_types=[
          pltpu.VMEM((CH,), jnp.int32),
          pltpu.VMEM((CH,), jnp.int32),
          pltpu.VMEM((CH,), jnp.int32),
          pltpu.VMEM((CH,), jnp.int32),
          pltpu.VMEM((CH,), jnp.int32),
          pltpu.VMEM((CH,), jnp.int32),
          pltpu.VMEM((CH,), jnp.int32),
          pltpu.VMEM((CH, L), jnp.float32),
          pltpu.VMEM((CHR, L), jnp.float32),
          pltpu.SemaphoreType.DMA,
      ],
      compiler_params=pltpu.CompilerParams(needs_layout_passes=False),
  )
  return f(users, items, Gu_r, Gi_r, ui1)


_T = 64


def _leaky(x):
  return jnp.where(x >= 0, x, 0.01 * x)


def _tc_body(gu_ref, gh_ref, gmi_ref, wa_ref, wb_ref, w2_ref, b1_ref, b2_ref,
             ones_ref, xui_ref, gs_ref):
  wa = wa_ref[...]
  wb = wb_ref[...]
  w2 = w2_ref[...]
  b1 = b1_ref[...]
  b2 = b2_ref[...]

  gu = gu_ref[...]
  au = jnp.dot(gu, wa, preferred_element_type=jnp.float32)

  gh = gh_ref[...].reshape(H * _T, L)
  hi = jnp.dot(gh, wb, preferred_element_type=jnp.float32)
  h1 = hi.reshape(H, _T, L) + (au + b1)[None]
  hbar = jnp.mean(_leaky(h1), axis=0)
  gs = jnp.dot(hbar, w2, preferred_element_type=jnp.float32) + b2

  ai = jnp.dot(gmi_ref[...], wb, preferred_element_type=jnp.float32)
  gl = _leaky(au + ai + b1)
  gui = jnp.dot(gl, w2, preferred_element_type=jnp.float32) + b2

  gs_ref[...] = gs
  xui_ref[...] = jnp.dot(gs * gui, ones_ref[...],
                         preferred_element_type=jnp.float32)


def _tc_mlp(gu, gihist, gmi, wa8, wb8, w28, b1t, b2t, ones8):
  grid = (B // R) // _T
  wspec = pl.BlockSpec((L, L), lambda i: (0, 0))
  bspec = pl.BlockSpec((1, L), lambda i: (0, 0))
  return pl.pallas_call(
      _tc_body,
      grid=(grid,),
      in_specs=[
          pl.BlockSpec((_T, L), lambda i: (i, 0)),
          pl.BlockSpec((H, _T, L), lambda i: (0, i, 0)),
          pl.BlockSpec((_T, L), lambda i: (i, 0)),
          wspec, wspec, wspec, bspec, bspec,
          pl.BlockSpec((L, R), lambda i: (0, 0)),
      ],
      out_specs=[
          pl.BlockSpec((_T, R), lambda i: (i, 0)),
          pl.BlockSpec((_T, L), lambda i: (i, 0)),
      ],
      out_shape=[
          jax.ShapeDtypeStruct((B // R, R), jnp.float32),
          jax.ShapeDtypeStruct((B // R, L), jnp.float32),
      ],
  )(gu, gihist, gmi, wa8, wb8, w28, b1t, b2t, ones8)


def kernel(users, items, Gu, Gi, W1, b1, W2, b2, ui):
  ui1 = ui[1]
  Gu_r = Gu.reshape(NUM_USERS // R, L)
  Gi_r = Gi.reshape(NUM_ITEMS // R, L)

  w1t = W1.T
  eye8 = jnp.eye(R, dtype=jnp.float32)
  wa8 = jnp.kron(eye8, w1t[:K])
  wb8 = jnp.kron(eye8, w1t[K:])
  w28 = jnp.kron(eye8, W2.T)
  b1t = jnp.tile(b1, R).reshape(1, L)
  b2t = jnp.tile(b2, R).reshape(1, L)
  ones8 = jnp.kron(eye8, jnp.ones((K, 1), jnp.float32))

  gu_p, gihist_p, gmi_p = _sc_gather(users, items, Gu_r, Gi_r, ui1)
  xui_p, gs_p = _tc_mlp(gu_p, gihist_p, gmi_p, wa8, wb8, w28, b1t, b2t, ones8)
  return (xui_p.reshape(B), gs_p.reshape(B, K), gmi_p.reshape(B, K))


# speedup vs baseline: 1.1297x; 1.1297x over previous
import jax
import jax.numpy as jnp
from jax import lax
from jax.experimental import pallas as pl
from jax.experimental.pallas import tpu as pltpu
from jax.experimental.pallas import tpu_sc as plsc

NUM_USERS = 100000
NUM_ITEMS = 1000000
K = 16
H = 20
B = 4096

NC = 2
NS = 16
NW = NC * NS
CH = B // NW
E = CH * H


def _sc_body(users_h, items_h, uiflat_h, *rest):
  gu_rows = rest[:K]
  gi_rows = rest[K:2 * K]
  rest = rest[2 * K:]
  guT_o, gihT_o, gmiT_o = rest[:3]
  rest = rest[3:]
  users_v, items_v, uif, histids = rest[:4]
  rest = rest[4:]
  ghb = rest[:K]
  gub = rest[K:2 * K]
  gmb = rest[2 * K:3 * K]
  sem_u, sem_g, sem_m, sem_h, sem_s = rest[3 * K:]

  wid = lax.axis_index("s") * NC + lax.axis_index("c")
  base = wid * CH

  pltpu.sync_copy(users_h.at[pl.ds(base, CH)], users_v)
  pltpu.sync_copy(items_h.at[pl.ds(base, CH)], items_v)

  gu_copies = [
      pltpu.async_copy(gu_rows[c].at[users_v], gub[c], sem_g)
      for c in range(K)
  ]
  gmi_copies = [
      pltpu.async_copy(gi_rows[c].at[items_v], gmb[c], sem_m)
      for c in range(K)
  ]

  def jblk(j, c0):
    def tblk(t, c1):
      u16 = users_v[pl.ds(t * 16, 16)]
      f = u16 * H + j
      flat = (lax.shift_right_logical(f, 7) * 256 + 128
              + jnp.bitwise_and(f, 127))
      uif[pl.ds(j * CH + t * 16, 16)] = flat
      return c1
    return lax.fori_loop(0, CH // 16, tblk, c0, unroll=True)
  lax.fori_loop(0, H, jblk, 0)

  pltpu.async_copy(uiflat_h.at[uif], histids, sem_u).wait()

  hist_copies = [
      pltpu.async_copy(gi_rows[c].at[histids], ghb[c], sem_h)
      for c in range(K)
  ]

  stores = []
  for c in range(K):
    gu_copies[c].wait()
    stores.append(pltpu.async_copy(gub[c], guT_o.at[c, pl.ds(base, CH)], sem_s))
  for c in range(K):
    gmi_copies[c].wait()
    stores.append(pltpu.async_copy(gmb[c], gmiT_o.at[c, pl.ds(base, CH)], sem_s))
  for c in range(K):
    hist_copies[c].wait()
    stores.append(pltpu.async_copy(ghb[c], gihT_o.at[c, pl.ds(wid * E, E)], sem_s))
  for st in stores:
    st.wait()


def _sc_gather(users, items, gu_rows, gi_rows, uiflat):
  mesh = plsc.VectorSubcoreMesh(core_axis_name="c", subcore_axis_name="s")
  f = pl.kernel(
      _sc_body,
      out_type=[
          jax.ShapeDtypeStruct((K, B), jnp.float32),
          jax.ShapeDtypeStruct((K, B * H), jnp.float32),
          jax.ShapeDtypeStruct((K, B), jnp.float32),
      ],
      mesh=mesh,
      scratch_types=(
          [
              pltpu.VMEM((CH,), jnp.int32),
              pltpu.VMEM((CH,), jnp.int32),
              pltpu.VMEM((E,), jnp.int32),
              pltpu.VMEM((E,), jnp.int32),
          ]
          + [pltpu.VMEM((E,), jnp.float32) for _ in range(K)]
          + [pltpu.VMEM((CH,), jnp.float32) for _ in range(K)]
          + [pltpu.VMEM((CH,), jnp.float32) for _ in range(K)]
          + [pltpu.SemaphoreType.DMA] * 5
      ),
  )
  return f(users, items, uiflat, *gu_rows, *gi_rows)


G = 8
TB = G * CH
TE = G * E


def _leaky(x):
  return jnp.where(x >= 0, x, 0.01 * x)


def _tc_body(gu_ref, gh_ref, gmi_ref, w1a_ref, w1b_ref, w2_ref, b1_ref, b2_ref,
             xui_ref, gs_ref):
  w1a = w1a_ref[...]
  w1b = w1b_ref[...]
  w2 = w2_ref[...]
  b1 = b1_ref[...]
  b2 = b2_ref[...]

  gu = gu_ref[...]
  au = jnp.dot(w1a, gu, preferred_element_type=jnp.float32)

  hi = jnp.dot(w1b, gh_ref[...], preferred_element_type=jnp.float32)
  h1 = (hi.reshape(K, G, H, CH)
        + (au + b1).reshape(K, G, 1, CH))
  hbar = jnp.mean(_leaky(h1), axis=2).reshape(K, TB)
  gs = jnp.dot(w2, hbar, preferred_element_type=jnp.float32) + b2

  ai = jnp.dot(w1b, gmi_ref[...], preferred_element_type=jnp.float32)
  gl = _leaky(au + ai + b1)
  gui = jnp.dot(w2, gl, preferred_element_type=jnp.float32) + b2

  gs_ref[...] = gs
  xui_ref[...] = jnp.sum(gs * gui, axis=0, keepdims=True)


def _tc_mlp(guT, gihT, gmiT, w1a, w1b, w2, b1c, b2c):
  grid = NW // G
  wspec = pl.BlockSpec((K, K), lambda i: (0, 0))
  bspec = pl.BlockSpec((K, 1), lambda i: (0, 0))
  return pl.pallas_call(
      _tc_body,
      grid=(grid,),
      in_specs=[
          pl.BlockSpec((K, TB), lambda i: (0, i)),
          pl.BlockSpec((K, TE), lambda i: (0, i)),
          pl.BlockSpec((K, TB), lambda i: (0, i)),
          wspec, wspec, wspec, bspec, bspec,
      ],
      out_specs=[
          pl.BlockSpec((1, TB), lambda i: (0, i)),
          pl.BlockSpec((K, TB), lambda i: (0, i)),
      ],
      out_shape=[
          jax.ShapeDtypeStruct((1, B), jnp.float32),
          jax.ShapeDtypeStruct((K, B), jnp.float32),
      ],
  )(guT, gihT, gmiT, w1a, w1b, w2, b1c, b2c)


def kernel(users, items, Gu, Gi, W1, b1, W2, b2, ui):
  gu_rows = [Gu[:, c] for c in range(K)]
  gi_rows = [Gi[:, c] for c in range(K)]
  uiflat = (ui.reshape(2, NUM_USERS * H // 128, 128)
            .transpose(1, 0, 2)
            .reshape(2 * NUM_USERS * H))

  w1a = W1[:, :K]
  w1b = W1[:, K:]
  b1c = b1.reshape(K, 1)
  b2c = b2.reshape(K, 1)

  guT, gihT, gmiT = _sc_gather(users, items, gu_rows, gi_rows, uiflat)
  xuiT, gsT = _tc_mlp(guT, gihT, gmiT, w1a, w1b, W2, b1c, b2c)
  return (xuiT.reshape(B), gsT.T, gmiT.T)
